# Initial kernel scaffold; baseline (speedup 1.0000x reference)
import jax
import jax.numpy as jnp
from jax.experimental import pallas as pl


def kernel(x, emb_table):
    raise NotImplementedError("write your pallas kernel here")


# speedup vs baseline: 1.7038x; 1.7038x over previous
import jax
import jax.numpy as jnp
from jax.experimental import pallas as pl


def _add_kernel(x_ref, emb_ref, out_ref):
    out_ref[...] = x_ref[...] + emb_ref[...][None, :, :]


def kernel(x, emb_table):
    batch, seq, d = x.shape
    sb = 512
    n_seq = seq // sb

    return pl.pallas_call(
        _add_kernel,
        grid=(n_seq, batch),
        in_specs=[
            pl.BlockSpec((1, sb, d), lambda s, b: (b, s, 0)),
            pl.BlockSpec((sb, d), lambda s, b: (s, 0)),
        ],
        out_specs=pl.BlockSpec((1, sb, d), lambda s, b: (b, s, 0)),
        out_shape=jax.ShapeDtypeStruct((batch, seq, d), x.dtype),
    )(x, emb_table)


# speedup vs baseline: 1.8830x; 1.1052x over previous
import jax
import jax.numpy as jnp
from jax.experimental import pallas as pl


def _add_kernel(x_ref, emb_ref, out_ref):
    out_ref[...] = x_ref[...] + emb_ref[...][None, :, :]


def kernel(x, emb_table):
    batch, seq, d = x.shape
    sb = 1024
    n_seq = seq // sb

    return pl.pallas_call(
        _add_kernel,
        grid=(n_seq, batch),
        in_specs=[
            pl.BlockSpec((1, sb, d), lambda s, b: (b, s, 0)),
            pl.BlockSpec((sb, d), lambda s, b: (s, 0)),
        ],
        out_specs=pl.BlockSpec((1, sb, d), lambda s, b: (b, s, 0)),
        out_shape=jax.ShapeDtypeStruct((batch, seq, d), x.dtype),
    )(x, emb_table)


# speedup vs baseline: 1.9946x; 1.0592x over previous
import jax
import jax.numpy as jnp
from jax.experimental import pallas as pl


def _add_kernel(x_ref, emb_ref, out_ref):
    out_ref[...] = x_ref[...] + emb_ref[...][None, :, :]


def kernel(x, emb_table):
    batch, seq, d = x.shape
    sb = 2048
    n_seq = seq // sb

    return pl.pallas_call(
        _add_kernel,
        grid=(n_seq, batch),
        in_specs=[
            pl.BlockSpec((1, sb, d), lambda s, b: (b, s, 0)),
            pl.BlockSpec((sb, d), lambda s, b: (s, 0)),
        ],
        out_specs=pl.BlockSpec((1, sb, d), lambda s, b: (b, s, 0)),
        out_shape=jax.ShapeDtypeStruct((batch, seq, d), x.dtype),
    )(x, emb_table)
